# baseline (speedup 1.0000x reference)
import jax
import jax.numpy as jnp
from jax.experimental import pallas as pl
from jax.experimental.pallas import tpu as pltpu
from jax.experimental.pallas import tpu_sc as plsc


def _sc_gather(emb, idx_flat, window=128):
    n_tok = idx_flat.shape[0]
    full_e = emb.shape[1]
    split = full_e // 128
    emb = emb.reshape(-1, 128)
    idx_flat = (
        idx_flat[:, None] * split
        + jnp.arange(split, dtype=jnp.int32)[None, :]
    ).reshape(-1)
    n = idx_flat.shape[0]
    e_dim = 128
    idx2 = idx_flat.reshape(1, n)
    mesh = plsc.VectorSubcoreMesh(core_axis_name="core", subcore_axis_name="subcore")

    @pl.kernel(out_type=jax.ShapeDtypeStruct((n, e_dim), emb.dtype), mesh=mesh)
    def gather_kernel(emb_hbm, i_hbm, o_hbm):
        def body(i_vmem, o_vmem):
            pltpu.sync_copy(emb_hbm.at[i_vmem.at[0]], o_vmem)

        pltpu.emit_pipeline(
            body,
            grid=(n // window,),
            in_specs=[pl.BlockSpec((1, window), lambda i: (0, i))],
            out_specs=[pl.BlockSpec((window, e_dim), lambda i: (i, 0))],
            core_axis_name=("core", "subcore"),
            dimension_semantics=(pltpu.PARALLEL,),
        )(i_hbm, o_hbm)

    return gather_kernel(emb, idx2).reshape(n_tok, full_e)


_DN_T = (((1,), (1,)), ((), ()))


def _in_gates(lhs_bf, w_raw, bias2):
    n = lhs_bf.shape[0]
    f = w_raw.shape[0]

    def body(l_ref, w_ref, b_ref, o_ref):
        o_ref[...] = (
            jax.lax.dot_general(
                l_ref[...], w_ref[...].astype(jnp.bfloat16), _DN_T,
                preferred_element_type=jnp.float32,
            )
            + b_ref[...]
        )

    return pl.pallas_call(
        body,
        out_shape=jax.ShapeDtypeStruct((n, f), jnp.float32),
    )(lhs_bf, w_raw, bias2)


def _gates(g, c, h_dim):
    gi = jax.nn.sigmoid(g[:, :h_dim])
    gf = jax.nn.sigmoid(g[:, h_dim:2 * h_dim])
    gg = jnp.tanh(g[:, 2 * h_dim:3 * h_dim])
    go = jax.nn.sigmoid(g[:, 3 * h_dim:])
    c = gf * c + gi * gg
    h = go * jnp.tanh(c)
    return h, c


def _wavefront(xg0, whh0_t, wih1_t, whh1_t, bias1, s, bn):
    f4 = xg0.shape[1]
    h_dim = f4 // 4
    bf = jnp.bfloat16

    def body(xg_ref, w00_ref, w10_ref, w11_ref, b1_ref,
             hseq_ref, h0_ref, c0_ref, h1_ref, c1_ref):
        def lay1_pre(h0b, h1b):
            return (
                jnp.dot(h0b, w10_ref[...], preferred_element_type=jnp.float32)
                + jnp.dot(h1b, w11_ref[...], preferred_element_type=jnp.float32)
                + b1_ref[...]
            )

        def step(k, carry):
            h0b, h1b, c0, c1 = carry
            g0 = xg_ref[pl.ds(k * bn, bn), :] + jnp.dot(
                h0b, w00_ref[...], preferred_element_type=jnp.float32)
            g1 = lay1_pre(h0b, h1b)
            h0, c0 = _gates(g0, c0, h_dim)
            h1, c1n = _gates(g1, c1, h_dim)
            h1b_n = jnp.where(k > 0, h1, 0.0).astype(bf)
            c1 = jnp.where(k > 0, c1n, c1)
            hseq_ref[jnp.maximum(k - 1, 0)] = h1.astype(bf)

            @pl.when(k == s - 1)
            def _():
                h0_ref[...] = h0
                c0_ref[...] = c0

            return (h0.astype(bf), h1b_n, c0, c1)

        zb = jnp.zeros((bn, h_dim), bf)
        zf = jnp.zeros((bn, h_dim), jnp.float32)
        h0b, h1b, c0, c1 = jax.lax.fori_loop(0, s, step, (zb, zb, zf, zf))

        g1 = lay1_pre(h0b, h1b)
        h1, c1 = _gates(g1, c1, h_dim)
        hseq_ref[s - 1] = h1.astype(bf)
        h1_ref[...] = h1
        c1_ref[...] = c1

    return pl.pallas_call(
        body,
        out_shape=[
            jax.ShapeDtypeStruct((s, bn, h_dim), bf),
            jax.ShapeDtypeStruct((bn, h_dim), jnp.float32),
            jax.ShapeDtypeStruct((bn, h_dim), jnp.float32),
            jax.ShapeDtypeStruct((bn, h_dim), jnp.float32),
            jax.ShapeDtypeStruct((bn, h_dim), jnp.float32),
        ],
    )(xg0, whh0_t, wih1_t, whh1_t, bias1)


def _fc(lhs_bf, w_raw, bias2, vt=1280):
    n, h_dim = lhs_bf.shape
    v = w_raw.shape[0]

    def body(l_ref, w_ref, b_ref, o_ref):
        o_ref[...] = (
            jax.lax.dot_general(
                l_ref[...], w_ref[...].astype(jnp.bfloat16), _DN_T,
                preferred_element_type=jnp.float32,
            )
            + b_ref[...]
        )

    return pl.pallas_call(
        body,
        grid=(v // vt,),
        in_specs=[
            pl.BlockSpec((n, h_dim), lambda j: (0, 0)),
            pl.BlockSpec((vt, h_dim), lambda j: (j, 0)),
            pl.BlockSpec((1, vt), lambda j: (0, j)),
        ],
        out_specs=pl.BlockSpec((n, vt), lambda j: (0, j)),
        out_shape=jax.ShapeDtypeStruct((n, v), jnp.float32),
    )(lhs_bf, w_raw, bias2)


def kernel(x, emb, W_ih0, W_hh0, b_ih0, b_hh0, W_ih1, W_hh1, b_ih1, b_hh1, fc_W, fc_b):
    bn, s = x.shape
    v, e_dim = emb.shape
    h_dim = W_hh0.shape[1]
    n = bn * s
    bf = jnp.bfloat16

    idx_flat = x.T.reshape(-1).astype(jnp.int32)
    e_sb = _sc_gather(emb, idx_flat)

    xg0 = _in_gates(e_sb.astype(bf), W_ih0, (b_ih0 + b_hh0).reshape(1, -1))

    h1seq, h0, c0, h1, c1 = _wavefront(
        xg0,
        W_hh0.astype(bf).T,
        W_ih1.astype(bf).T,
        W_hh1.astype(bf).T,
        (b_ih1 + b_hh1).reshape(1, -1),
        s, bn,
    )

    o1_bs = jnp.swapaxes(h1seq, 0, 1).reshape(n, h_dim)
    logits = _fc(
        o1_bs, fc_W, fc_b.reshape(1, -1)
    ).reshape(bn, s, v)

    h_out = jnp.stack([h0, h1], axis=0)
    c_out = jnp.stack([c0, c1], axis=0)
    return (logits, h_out, c_out)


# speedup vs baseline: 1.2755x; 1.2755x over previous
import jax
import jax.numpy as jnp
from jax.experimental import pallas as pl
from jax.experimental.pallas import tpu as pltpu
from jax.experimental.pallas import tpu_sc as plsc


def _sc_gather(emb, idx_flat, window=128):
    n_tok = idx_flat.shape[0]
    full_e = emb.shape[1]
    split = full_e // 128
    emb = emb.reshape(-1, 128)
    idx_flat = (
        idx_flat[:, None] * split
        + jnp.arange(split, dtype=jnp.int32)[None, :]
    ).reshape(-1)
    n = idx_flat.shape[0]
    e_dim = 128
    idx2 = idx_flat.reshape(1, n)
    mesh = plsc.VectorSubcoreMesh(core_axis_name="core", subcore_axis_name="subcore")

    @pl.kernel(out_type=jax.ShapeDtypeStruct((n, e_dim), emb.dtype), mesh=mesh)
    def gather_kernel(emb_hbm, i_hbm, o_hbm):
        def body(i_vmem, o_vmem):
            pltpu.sync_copy(emb_hbm.at[i_vmem.at[0]], o_vmem)

        pltpu.emit_pipeline(
            body,
            grid=(n // window,),
            in_specs=[pl.BlockSpec((1, window), lambda i: (0, i))],
            out_specs=[pl.BlockSpec((window, e_dim), lambda i: (i, 0))],
            core_axis_name=("core", "subcore"),
            dimension_semantics=(pltpu.PARALLEL,),
        )(i_hbm, o_hbm)

    return gather_kernel(emb, idx2).reshape(n_tok, full_e)


_DN_T = (((1,), (1,)), ((), ()))
_UNROLL = 4


def _lstm_layer(lhs, w_ih_raw, bias2, whh_t_bf, s, bn):
    n = lhs.shape[0]
    f4 = w_ih_raw.shape[0]
    h_dim = f4 // 4

    def body(l_ref, wih_ref, b_ref, whh_ref, hseq_ref, h_ref, c_ref, xg_s):
        xg_s[...] = jax.lax.dot_general(
            l_ref[...].astype(jnp.bfloat16),
            wih_ref[...].astype(jnp.bfloat16), _DN_T,
            preferred_element_type=jnp.float32,
        ) + b_ref[...]

        def step(j, carry):
            h, c = carry
            for u in range(_UNROLL):
                t = j * _UNROLL + u
                g = xg_s[pl.ds(t * bn, bn), :] + jnp.dot(
                    h.astype(jnp.bfloat16), whh_ref[...],
                    preferred_element_type=jnp.float32,
                )
                gi = jax.nn.sigmoid(g[:, :h_dim])
                gf = jax.nn.sigmoid(g[:, h_dim:2 * h_dim])
                gg = jnp.tanh(g[:, 2 * h_dim:3 * h_dim])
                go = jax.nn.sigmoid(g[:, 3 * h_dim:])
                c = gf * c + gi * gg
                h = go * jnp.tanh(c)
                hseq_ref[t] = h.astype(jnp.bfloat16)
            return (h, c)

        zero = jnp.zeros((bn, h_dim), jnp.float32)
        h, c = jax.lax.fori_loop(0, s // _UNROLL, step, (zero, zero))
        h_ref[...] = h
        c_ref[...] = c

    return pl.pallas_call(
        body,
        out_shape=[
            jax.ShapeDtypeStruct((s, bn, h_dim), jnp.bfloat16),
            jax.ShapeDtypeStruct((bn, h_dim), jnp.float32),
            jax.ShapeDtypeStruct((bn, h_dim), jnp.float32),
        ],
        scratch_shapes=[
            pltpu.VMEM((n, f4), jnp.float32),
        ],
    )(lhs, w_ih_raw, bias2, whh_t_bf)


def _fc(lhs_bf, w_raw, bias2, vt=3200):
    n, h_dim = lhs_bf.shape
    v = w_raw.shape[0]

    def body(l_ref, w_ref, b_ref, o_ref):
        o_ref[...] = (
            jax.lax.dot_general(
                l_ref[...], w_ref[...].astype(jnp.bfloat16), _DN_T,
                preferred_element_type=jnp.float32,
            )
            + b_ref[...]
        )

    return pl.pallas_call(
        body,
        grid=(v // vt,),
        in_specs=[
            pl.BlockSpec((n, h_dim), lambda j: (0, 0)),
            pl.BlockSpec((vt, h_dim), lambda j: (j, 0)),
            pl.BlockSpec((1, vt), lambda j: (0, j)),
        ],
        out_specs=pl.BlockSpec((n, vt), lambda j: (0, j)),
        out_shape=jax.ShapeDtypeStruct((n, v), jnp.float32),
    )(lhs_bf, w_raw, bias2)


def kernel(x, emb, W_ih0, W_hh0, b_ih0, b_hh0, W_ih1, W_hh1, b_ih1, b_hh1, fc_W, fc_b):
    bn, s = x.shape
    v, e_dim = emb.shape
    h_dim = W_hh0.shape[1]
    n = bn * s
    bf = jnp.bfloat16

    idx_flat = x.T.reshape(-1).astype(jnp.int32)
    e_sb = _sc_gather(emb, idx_flat)

    h0seq, h0, c0 = _lstm_layer(
        e_sb, W_ih0, (b_ih0 + b_hh0).reshape(1, -1),
        W_hh0.astype(bf).T, s, bn,
    )

    h1seq, h1, c1 = _lstm_layer(
        h0seq.reshape(n, h_dim), W_ih1, (b_ih1 + b_hh1).reshape(1, -1),
        W_hh1.astype(bf).T, s, bn,
    )

    o1_bs = jnp.swapaxes(h1seq, 0, 1).reshape(n, h_dim)
    logits = _fc(
        o1_bs, fc_W, fc_b.reshape(1, -1)
    ).reshape(bn, s, v)

    h_out = jnp.stack([h0, h1], axis=0)
    c_out = jnp.stack([c0, c1], axis=0)
    return (logits, h_out, c_out)


# speedup vs baseline: 1.2762x; 1.0005x over previous
import jax
import jax.numpy as jnp
from jax.experimental import pallas as pl
from jax.experimental.pallas import tpu as pltpu
from jax.experimental.pallas import tpu_sc as plsc


def _sc_gather(emb, idx_flat, window=128):
    n_tok = idx_flat.shape[0]
    full_e = emb.shape[1]
    split = full_e // 128
    emb = emb.reshape(-1, 128)
    idx_flat = (
        idx_flat[:, None] * split
        + jnp.arange(split, dtype=jnp.int32)[None, :]
    ).reshape(-1)
    n = idx_flat.shape[0]
    e_dim = 128
    idx2 = idx_flat.reshape(1, n)
    mesh = plsc.VectorSubcoreMesh(core_axis_name="core", subcore_axis_name="subcore")

    @pl.kernel(out_type=jax.ShapeDtypeStruct((n, e_dim), emb.dtype), mesh=mesh)
    def gather_kernel(emb_hbm, i_hbm, o_hbm):
        def body(i_vmem, o_vmem):
            pltpu.sync_copy(emb_hbm.at[i_vmem.at[0]], o_vmem)

        pltpu.emit_pipeline(
            body,
            grid=(n // window,),
            in_specs=[pl.BlockSpec((1, window), lambda i: (0, i))],
            out_specs=[pl.BlockSpec((window, e_dim), lambda i: (i, 0))],
            core_axis_name=("core", "subcore"),
            dimension_semantics=(pltpu.PARALLEL,),
        )(i_hbm, o_hbm)

    return gather_kernel(emb, idx2).reshape(n_tok, full_e)


_DN_T = (((1,), (1,)), ((), ()))
_UNROLL = 8


def _lstm_layer(lhs, w_ih_raw, bias2, whh_t_bf, s, bn):
    n = lhs.shape[0]
    f4 = w_ih_raw.shape[0]
    h_dim = f4 // 4

    def body(l_ref, wih_ref, b_ref, whh_ref, hseq_ref, h_ref, c_ref, xg_s):
        xg_s[...] = jax.lax.dot_general(
            l_ref[...].astype(jnp.bfloat16),
            wih_ref[...].astype(jnp.bfloat16), _DN_T,
            preferred_element_type=jnp.float32,
        ) + b_ref[...]

        def step(j, carry):
            h, c = carry
            for u in range(_UNROLL):
                t = j * _UNROLL + u
                g = xg_s[pl.ds(t * bn, bn), :] + jnp.dot(
                    h.astype(jnp.bfloat16), whh_ref[...],
                    preferred_element_type=jnp.float32,
                )
                gi = jax.nn.sigmoid(g[:, :h_dim])
                gf = jax.nn.sigmoid(g[:, h_dim:2 * h_dim])
                gg = jnp.tanh(g[:, 2 * h_dim:3 * h_dim])
                go = jax.nn.sigmoid(g[:, 3 * h_dim:])
                c = gf * c + gi * gg
                h = go * jnp.tanh(c)
                hseq_ref[t] = h.astype(jnp.bfloat16)
            return (h, c)

        zero = jnp.zeros((bn, h_dim), jnp.float32)
        h, c = jax.lax.fori_loop(0, s // _UNROLL, step, (zero, zero))
        h_ref[...] = h
        c_ref[...] = c

    return pl.pallas_call(
        body,
        out_shape=[
            jax.ShapeDtypeStruct((s, bn, h_dim), jnp.bfloat16),
            jax.ShapeDtypeStruct((bn, h_dim), jnp.float32),
            jax.ShapeDtypeStruct((bn, h_dim), jnp.float32),
        ],
        scratch_shapes=[
            pltpu.VMEM((n, f4), jnp.float32),
        ],
    )(lhs, w_ih_raw, bias2, whh_t_bf)


def _fc(lhs_bf, w_raw, bias2, vt=3200):
    n, h_dim = lhs_bf.shape
    v = w_raw.shape[0]

    def body(l_ref, w_ref, b_ref, o_ref):
        o_ref[...] = (
            jax.lax.dot_general(
                l_ref[...], w_ref[...].astype(jnp.bfloat16), _DN_T,
                preferred_element_type=jnp.float32,
            )
            + b_ref[...]
        )

    return pl.pallas_call(
        body,
        grid=(v // vt,),
        in_specs=[
            pl.BlockSpec((n, h_dim), lambda j: (0, 0)),
            pl.BlockSpec((vt, h_dim), lambda j: (j, 0)),
            pl.BlockSpec((1, vt), lambda j: (0, j)),
        ],
        out_specs=pl.BlockSpec((n, vt), lambda j: (0, j)),
        out_shape=jax.ShapeDtypeStruct((n, v), jnp.float32),
    )(lhs_bf, w_raw, bias2)


def kernel(x, emb, W_ih0, W_hh0, b_ih0, b_hh0, W_ih1, W_hh1, b_ih1, b_hh1, fc_W, fc_b):
    bn, s = x.shape
    v, e_dim = emb.shape
    h_dim = W_hh0.shape[1]
    n = bn * s
    bf = jnp.bfloat16

    idx_flat = x.T.reshape(-1).astype(jnp.int32)
    e_sb = _sc_gather(emb, idx_flat)

    h0seq, h0, c0 = _lstm_layer(
        e_sb, W_ih0, (b_ih0 + b_hh0).reshape(1, -1),
        W_hh0.astype(bf).T, s, bn,
    )

    h1seq, h1, c1 = _lstm_layer(
        h0seq.reshape(n, h_dim), W_ih1, (b_ih1 + b_hh1).reshape(1, -1),
        W_hh1.astype(bf).T, s, bn,
    )

    o1_bs = jnp.swapaxes(h1seq, 0, 1).reshape(n, h_dim)
    logits = _fc(
        o1_bs, fc_W, fc_b.reshape(1, -1)
    ).reshape(bn, s, v)

    h_out = jnp.stack([h0, h1], axis=0)
    c_out = jnp.stack([c0, c1], axis=0)
    return (logits, h_out, c_out)


# speedup vs baseline: 1.4557x; 1.1407x over previous
import jax
import jax.numpy as jnp
from jax.experimental import pallas as pl
from jax.experimental.pallas import tpu as pltpu
from jax.experimental.pallas import tpu_sc as plsc


def _sc_gather(emb, idx_flat):
    n_tok = idx_flat.shape[0]
    full_e = emb.shape[1]
    idx2 = idx_flat.reshape(1, n_tok)
    mesh = plsc.VectorSubcoreMesh(core_axis_name="core", subcore_axis_name="subcore")
    n_units = mesh.num_cores * mesh.num_subcores
    rows_per = n_tok // n_units

    @pl.kernel(
        out_type=jax.ShapeDtypeStruct((n_tok, full_e), emb.dtype),
        mesh=mesh,
        scratch_types=[
            pltpu.VMEM((1, n_tok), jnp.int32),
            pltpu.VMEM((rows_per, full_e), emb.dtype),
        ],
    )
    def gather_kernel(emb_hbm, i_hbm, o_hbm, idx_l, out_l):
        cid = jax.lax.axis_index("core")
        sid = jax.lax.axis_index("subcore")
        start = (cid * mesh.num_subcores + sid) * rows_per
        pltpu.sync_copy(i_hbm, idx_l)
        pltpu.sync_copy(emb_hbm.at[idx_l.at[0, pl.ds(start, rows_per)]], out_l)
        pltpu.sync_copy(out_l, o_hbm.at[pl.ds(start, rows_per)])

    return gather_kernel(emb, idx2)


_DN_T = (((1,), (1,)), ((), ()))
_UNROLL = 8


def _lstm_layer(lhs, w_ih_raw, bias2, whh_t_bf, s, bn):
    n = lhs.shape[0]
    f4 = w_ih_raw.shape[0]
    h_dim = f4 // 4

    def body(l_ref, wih_ref, b_ref, whh_ref, hseq_ref, h_ref, c_ref, xg_s):
        xg_s[...] = jax.lax.dot_general(
            l_ref[...].astype(jnp.bfloat16),
            wih_ref[...].astype(jnp.bfloat16), _DN_T,
            preferred_element_type=jnp.float32,
        ) + b_ref[...]

        def step(j, carry):
            h, c = carry
            for u in range(_UNROLL):
                t = j * _UNROLL + u
                g = xg_s[pl.ds(t * bn, bn), :] + jnp.dot(
                    h.astype(jnp.bfloat16), whh_ref[...],
                    preferred_element_type=jnp.float32,
                )
                gi = jax.nn.sigmoid(g[:, :h_dim])
                gf = jax.nn.sigmoid(g[:, h_dim:2 * h_dim])
                gg = jnp.tanh(g[:, 2 * h_dim:3 * h_dim])
                go = jax.nn.sigmoid(g[:, 3 * h_dim:])
                c = gf * c + gi * gg
                h = go * jnp.tanh(c)
                hseq_ref[t] = h.astype(jnp.bfloat16)
            return (h, c)

        zero = jnp.zeros((bn, h_dim), jnp.float32)
        h, c = jax.lax.fori_loop(0, s // _UNROLL, step, (zero, zero))
        h_ref[...] = h
        c_ref[...] = c

    return pl.pallas_call(
        body,
        out_shape=[
            jax.ShapeDtypeStruct((s, bn, h_dim), jnp.bfloat16),
            jax.ShapeDtypeStruct((bn, h_dim), jnp.float32),
            jax.ShapeDtypeStruct((bn, h_dim), jnp.float32),
        ],
        scratch_shapes=[
            pltpu.VMEM((n, f4), jnp.float32),
        ],
    )(lhs, w_ih_raw, bias2, whh_t_bf)


def _fc(lhs_bf, w_raw, bias2, vt=3200):
    n, h_dim = lhs_bf.shape
    v = w_raw.shape[0]

    def body(l_ref, w_ref, b_ref, o_ref):
        o_ref[...] = (
            jax.lax.dot_general(
                l_ref[...], w_ref[...].astype(jnp.bfloat16), _DN_T,
                preferred_element_type=jnp.float32,
            )
            + b_ref[...]
        )

    return pl.pallas_call(
        body,
        grid=(v // vt,),
        in_specs=[
            pl.BlockSpec((n, h_dim), lambda j: (0, 0)),
            pl.BlockSpec((vt, h_dim), lambda j: (j, 0)),
            pl.BlockSpec((1, vt), lambda j: (0, j)),
        ],
        out_specs=pl.BlockSpec((n, vt), lambda j: (0, j)),
        out_shape=jax.ShapeDtypeStruct((n, v), jnp.float32),
    )(lhs_bf, w_raw, bias2)


def kernel(x, emb, W_ih0, W_hh0, b_ih0, b_hh0, W_ih1, W_hh1, b_ih1, b_hh1, fc_W, fc_b):
    bn, s = x.shape
    v, e_dim = emb.shape
    h_dim = W_hh0.shape[1]
    n = bn * s
    bf = jnp.bfloat16

    idx_flat = x.T.reshape(-1).astype(jnp.int32)
    e_sb = _sc_gather(emb, idx_flat)

    h0seq, h0, c0 = _lstm_layer(
        e_sb, W_ih0, (b_ih0 + b_hh0).reshape(1, -1),
        W_hh0.astype(bf).T, s, bn,
    )

    h1seq, h1, c1 = _lstm_layer(
        h0seq.reshape(n, h_dim), W_ih1, (b_ih1 + b_hh1).reshape(1, -1),
        W_hh1.astype(bf).T, s, bn,
    )

    o1_bs = jnp.swapaxes(h1seq, 0, 1).reshape(n, h_dim)
    logits = _fc(
        o1_bs, fc_W, fc_b.reshape(1, -1)
    ).reshape(bn, s, v)

    h_out = jnp.stack([h0, h1], axis=0)
    c_out = jnp.stack([c0, c1], axis=0)
    return (logits, h_out, c_out)
